# baseline (speedup 1.0000x reference)
import functools

import numpy as np
import jax
import jax.numpy as jnp
from jax import lax
from jax.experimental import pallas as pl
from jax.experimental.pallas import tpu as pltpu
from jax.experimental.pallas import tpu_sc as plsc

_N_ATOMS = 100000
_NN = 16
_N_TYPES = 4
_N_DESC = 16
_K_MAX = 4
_R_C = 5.0

_NP = 102400
_NC = 2
_NS = 16
_NW = _NC * _NS
_APW = _NP // _NW
_CA = 640
_NCH = _APW // _CA


_PA = 6400


def _sc_gather_t(idx, posf, typ):
    mesh = plsc.VectorSubcoreMesh(core_axis_name="c", subcore_axis_name="s")

    @functools.partial(
        pl.kernel,
        mesh=mesh,
        compiler_params=pltpu.CompilerParams(use_tc_tiling_on_sc=False,
                                             needs_layout_passes=False),
        out_type=(jax.ShapeDtypeStruct((3, _NN, _NP), jnp.float32),
                  jax.ShapeDtypeStruct((_NC, 3, _N_ATOMS), jnp.float32)),
        scratch_types=[
            pltpu.VMEM((_N_ATOMS,), jnp.float32),
            pltpu.VMEM((_CA * _NN,), jnp.int32),
            pltpu.VMEM((_NN, _CA), jnp.float32),
        ],
    )
    def k(idx_hbm, posf_hbm, typ_hbm, out_hbm, tabs_hbm, tab_v, idx_v, out_v):
        sid = lax.axis_index("s")
        cid = lax.axis_index("c")
        wid = sid * _NC + cid
        iot16 = lax.iota(jnp.int32, 16) * _NN
        iot3 = lax.iota(jnp.int32, 16) * 3

        b0 = jnp.minimum(sid * _PA, _N_ATOMS - _PA)
        pltpu.sync_copy(posf_hbm.at[pl.ds(b0 * 3, _PA * 3)],
                        tab_v.at[pl.ds(0, _PA * 3)])
        pltpu.sync_copy(typ_hbm.at[pl.ds(b0, _PA)], idx_v.at[pl.ds(0, _PA)])
        for c in range(3):
            for r in range(_PA // _CA):
                @plsc.parallel_loop(0, _CA // 16, unroll=2)
                def pbody(g):
                    a = r * _CA + g * 16
                    vals = plsc.load_gather(tab_v, [a * 3 + iot3 + c])
                    if c == 0:
                        tv = idx_v[pl.ds(a, 16)]
                        xb = lax.bitcast_convert_type(vals, jnp.int32)
                        vals = lax.bitcast_convert_type(
                            (xb & ~jnp.int32(3)) | tv, jnp.float32)
                    out_v[c & 1, pl.ds(g * 16, 16)] = vals

                pltpu.sync_copy(
                    out_v.at[c & 1],
                    tabs_hbm.at[cid, c, pl.ds(b0 + r * _CA, _CA)])
        plsc.subcore_barrier()

        for c in range(3):
            pltpu.sync_copy(tabs_hbm.at[cid, c], tab_v)
            for ci in range(_NCH):
                col0 = jnp.minimum(wid * _APW + ci * _CA, _N_ATOMS - _CA)
                pltpu.sync_copy(idx_hbm.at[pl.ds(col0 * _NN, _CA * _NN)],
                                idx_v)

                @plsc.parallel_loop(0, _CA // 16, unroll=2)
                def body(g):
                    a0 = g * 16
                    ibase = a0 * _NN + iot16
                    nbrs = [plsc.load_gather(idx_v, [ibase + j])
                            for j in range(_NN)]
                    vals = [plsc.load_gather(tab_v, [nbrs[j]])
                            for j in range(_NN)]
                    for j in range(_NN):
                        out_v[j, pl.ds(a0, 16)] = vals[j]

                pltpu.sync_copy(out_v, out_hbm.at[c, :, pl.ds(col0, _CA)])

    return k(idx, posf, typ)


_BAT = 2048


def _tc_body3(cmp_ref, pos_ref, qt_ref, o_ref):
    c3 = cmp_ref[...]
    xj, yj, zj = c3[0], c3[1], c3[2]
    tj = lax.bitcast_convert_type(xj, jnp.int32) & 3
    pp = pos_ref[0]
    dx = xj - pp[0:1, :]
    dy = yj - pp[1:2, :]
    dz = zj - pp[2:3, :]
    ti = lax.bitcast_convert_type(pp[0:1, :], jnp.int32) & 3
    r2 = dx * dx + dy * dy + dz * dz
    r = jnp.sqrt(r2)
    u = r * (1.0 / _R_C)
    w = 2.0 * u - 1.0
    w2 = w * w
    s = w * (1.570792378137 + w2 * (-0.645905999200 + w2 *
             (0.079464822790 + w2 * -0.004352781890)))
    hfc = jnp.where(r < _R_C, 0.25 - 0.25 * s, 0.0)
    x = 2.0 * (u - 1.0) * (u - 1.0) - 1.0
    t2 = 2.0 * x * x - 1.0
    t3 = 2.0 * x * t2 - x
    fns = [2.0 * hfc, (x + 1.0) * hfc, (t2 + 1.0) * hfc, (t3 + 1.0) * hfc]
    rows = []
    for tp in range(_N_TYPES):
        m = (tj == tp)
        for kk in range(_K_MAX):
            rows.append(jnp.where(m, fns[kk], 0.0))
    phi = jnp.concatenate(rows, axis=0).astype(jnp.bfloat16)
    g4 = lax.dot_general(qt_ref[...], phi, (((1,), (0,)), ((), ())),
                         preferred_element_type=jnp.float32)
    acc = jnp.zeros((_NN, _BAT), jnp.float32)
    for t in range(_N_TYPES):
        m = (ti == t)
        acc = acc + jnp.where(m, g4[_N_DESC * t:_N_DESC * (t + 1), :], 0.0)
    o_ref[...] = acc.T


def _tc_math3(cmps, tabs, qt):
    return pl.pallas_call(
        _tc_body3,
        grid=(pl.cdiv(_N_ATOMS, _BAT),),
        in_specs=[
            pl.BlockSpec((3, _NN, _BAT), lambda i: (0, 0, i)),
            pl.BlockSpec((1, 3, _BAT), lambda i: (0, 0, i)),
            pl.BlockSpec((64, 256), lambda i: (0, 0)),
        ],
        out_specs=pl.BlockSpec((_BAT, _NN), lambda i: (i, 0)),
        out_shape=jax.ShapeDtypeStruct((_N_ATOMS, _NN), jnp.float32),
    )(cmps, tabs, qt)


def _qt_const(c_table):
    base = jnp.transpose(c_table.astype(jnp.float32), (0, 2, 1, 3))
    qt16 = base.reshape(64, 16)
    qt = jnp.broadcast_to(qt16[:, :, None], (64, 16, 16)).reshape(64, 256)
    return qt.astype(jnp.bfloat16)


def kernel(types, positions, radial_neighbors, c_table):
    posf = positions.astype(jnp.float32).reshape(-1)
    ti32 = types.astype(jnp.int32)
    idx = radial_neighbors.astype(jnp.int32).reshape(-1)
    cmps, tabs = _sc_gather_t(idx, posf, ti32)
    return _tc_math3(cmps, tabs, _qt_const(c_table))


# speedup vs baseline: 1.4551x; 1.4551x over previous
import functools

import numpy as np
import jax
import jax.numpy as jnp
from jax import lax
from jax.experimental import pallas as pl
from jax.experimental.pallas import tpu as pltpu
from jax.experimental.pallas import tpu_sc as plsc

_N_ATOMS = 100000
_NN = 16
_N_TYPES = 4
_N_DESC = 16
_K_MAX = 4
_R_C = 5.0

_NP = 102400
_NC = 2
_NS = 16
_NW = _NC * _NS
_APW = _NP // _NW
_CA = 640
_NCH = _APW // _CA


def _sc_gather_t(idxt, tabs):
    mesh = plsc.VectorSubcoreMesh(core_axis_name="c", subcore_axis_name="s")

    @functools.partial(
        pl.kernel,
        mesh=mesh,
        compiler_params=pltpu.CompilerParams(use_tc_tiling_on_sc=False,
                                             needs_layout_passes=False),
        out_type=jax.ShapeDtypeStruct((3, _NN, _NP), jnp.float32),
        scratch_types=[
            pltpu.VMEM((_N_ATOMS,), jnp.float32),
            pltpu.VMEM((_NN, _CA), jnp.int32),
            pltpu.VMEM((_NN, _CA), jnp.float32),
        ],
    )
    def k(idxt_hbm, tabs_hbm, out_hbm, tab_v, idxt_v, out_v):
        wid = lax.axis_index("s") * _NC + lax.axis_index("c")
        for c in range(3):
            pltpu.sync_copy(tabs_hbm.at[c], tab_v)
            for ci in range(_NCH):
                col0 = jnp.minimum(wid * _APW + ci * _CA, _N_ATOMS - _CA)
                pltpu.sync_copy(idxt_hbm.at[:, pl.ds(col0, _CA)], idxt_v)

                @plsc.parallel_loop(0, _CA // 16, unroll=2)
                def body(g):
                    a0 = g * 16
                    for j in range(_NN):
                        nbr = idxt_v[j, pl.ds(a0, 16)]
                        out_v[j, pl.ds(a0, 16)] = plsc.load_gather(tab_v,
                                                                   [nbr])

                pltpu.sync_copy(out_v, out_hbm.at[c, :, pl.ds(col0, _CA)])

    return k(idxt, tabs)


_BAT = 2048


def _tc_body3(cmp_ref, pos_ref, qt_ref, o_ref):
    c3 = cmp_ref[...]
    xj, yj, zj = c3[0], c3[1], c3[2]
    tj = lax.bitcast_convert_type(xj, jnp.int32) & 3
    pp = pos_ref[...]
    dx = xj - pp[0:1, :]
    dy = yj - pp[1:2, :]
    dz = zj - pp[2:3, :]
    ti = lax.bitcast_convert_type(pp[0:1, :], jnp.int32) & 3
    r2 = dx * dx + dy * dy + dz * dz
    r = jnp.sqrt(r2)
    u = r * (1.0 / _R_C)
    w = 2.0 * u - 1.0
    w2 = w * w
    s = w * (1.570792378137 + w2 * (-0.645905999200 + w2 *
             (0.079464822790 + w2 * -0.004352781890)))
    hfc = jnp.where(r < _R_C, 0.25 - 0.25 * s, 0.0)
    x = 2.0 * (u - 1.0) * (u - 1.0) - 1.0
    t2 = 2.0 * x * x - 1.0
    t3 = 2.0 * x * t2 - x
    fns = [2.0 * hfc, (x + 1.0) * hfc, (t2 + 1.0) * hfc, (t3 + 1.0) * hfc]
    rows = []
    for tp in range(_N_TYPES):
        m = (tj == tp)
        for kk in range(_K_MAX):
            rows.append(jnp.where(m, fns[kk], 0.0))
    phi = jnp.concatenate(rows, axis=0).astype(jnp.bfloat16)
    g4 = lax.dot_general(qt_ref[...], phi, (((1,), (0,)), ((), ())),
                         preferred_element_type=jnp.float32)
    acc = jnp.zeros((_NN, _BAT), jnp.float32)
    for t in range(_N_TYPES):
        m = (ti == t)
        acc = acc + jnp.where(m, g4[_N_DESC * t:_N_DESC * (t + 1), :], 0.0)
    o_ref[...] = acc.T


def _tc_math3(cmps, tabs, qt):
    return pl.pallas_call(
        _tc_body3,
        grid=(pl.cdiv(_N_ATOMS, _BAT),),
        in_specs=[
            pl.BlockSpec((3, _NN, _BAT), lambda i: (0, 0, i)),
            pl.BlockSpec((3, _BAT), lambda i: (0, i)),
            pl.BlockSpec((64, 256), lambda i: (0, 0)),
        ],
        out_specs=pl.BlockSpec((_BAT, _NN), lambda i: (i, 0)),
        out_shape=jax.ShapeDtypeStruct((_N_ATOMS, _NN), jnp.float32),
    )(cmps, tabs, qt)


def _qt_const(c_table):
    base = jnp.transpose(c_table.astype(jnp.float32), (0, 2, 1, 3))
    qt16 = base.reshape(64, 16)
    qt = jnp.broadcast_to(qt16[:, :, None], (64, 16, 16)).reshape(64, 256)
    return qt.astype(jnp.bfloat16)


def kernel(types, positions, radial_neighbors, c_table):
    pos = positions.astype(jnp.float32)
    ti32 = types.astype(jnp.int32)
    xbits = lax.bitcast_convert_type(pos[:, 0], jnp.int32)
    xenc = lax.bitcast_convert_type((xbits & ~jnp.int32(3)) | ti32,
                                    jnp.float32)
    tabs = jnp.stack([xenc, pos[:, 1], pos[:, 2]], axis=0)
    idxt = radial_neighbors.astype(jnp.int32).T
    cmps = _sc_gather_t(idxt, tabs)
    return _tc_math3(cmps, tabs, _qt_const(c_table))
